# Initial kernel scaffold; baseline (speedup 1.0000x reference)
import jax
import jax.numpy as jnp
from jax.experimental import pallas as pl


def kernel(ids, conditions, pos_ids, ids_table, pos_table, cond_W, cond_b, ln_scale, ln_bias):
    raise NotImplementedError("write your pallas kernel here")


# speedup vs baseline: 1.9694x; 1.9694x over previous
import functools

import jax
import jax.numpy as jnp
from jax import lax
from jax.experimental import pallas as pl
from jax.experimental.pallas import tpu as pltpu
from jax.experimental.pallas import tpu_sc as plsc

_VOCAB = 100000
_MAXLEN = 2048
_EMB = 768
_COND = 128
_B = 4
_S = 2048
_EPS = 1e-12

_NTOK = _B * _S
_NC = 2
_NS = 16
_NW = _NC * _NS
_PER_W = _NTOK // _NW
_C = 64
_NCHUNK = _PER_W // _C
_LANES = 16


def _sc_gather_sum(ids_flat, pos_flat, ids_table, pos_table):
    mesh = plsc.VectorSubcoreMesh(core_axis_name="c", subcore_axis_name="s")

    @functools.partial(
        pl.kernel,
        mesh=mesh,
        out_type=jax.ShapeDtypeStruct((_NTOK, _EMB), jnp.float32),
        scratch_types=[
            pltpu.VMEM((_C,), jnp.int32),
            pltpu.VMEM((_C,), jnp.int32),
            pltpu.VMEM((_C, _EMB), jnp.float32),
            pltpu.VMEM((_C, _EMB), jnp.float32),
            pltpu.SemaphoreType.DMA,
            pltpu.SemaphoreType.DMA,
        ],
    )
    def sc_kernel(ids_hbm, pos_hbm, idtab_hbm, postab_hbm, out_hbm,
                  idx_i, idx_p, rows_i, rows_p, sem_i, sem_p):
        wid = lax.axis_index("s") * _NC + lax.axis_index("c")
        base = wid * _PER_W

        def chunk_body(ci, carry):
            start = base + ci * _C
            pltpu.sync_copy(ids_hbm.at[pl.ds(start, _C)], idx_i)
            pltpu.sync_copy(pos_hbm.at[pl.ds(start, _C)], idx_p)
            cp_i = pltpu.async_copy(idtab_hbm.at[idx_i], rows_i, sem_i)
            cp_p = pltpu.async_copy(postab_hbm.at[idx_p], rows_p, sem_p)
            cp_i.wait()
            cp_p.wait()

            def row_body(r, c2):
                for v in range(_EMB // _LANES):
                    sl = pl.ds(v * _LANES, _LANES)
                    rows_i[r, sl] = rows_i[r, sl] + rows_p[r, sl]
                return c2

            lax.fori_loop(0, _C, row_body, 0)
            pltpu.sync_copy(rows_i, out_hbm.at[pl.ds(start, _C)])
            return carry

        lax.fori_loop(0, _NCHUNK, chunk_body, 0)

    return sc_kernel(ids_flat, pos_flat, ids_table, pos_table)


_TBLK = 512


def _tc_body(g_ref, cond_ref, w_ref, b_ref, scale_ref, bias_ref, o_ref):
    proj = jnp.dot(cond_ref[...], w_ref[...], preferred_element_type=jnp.float32)
    x = g_ref[...] + jnp.tanh(proj + b_ref[...])
    mu = jnp.mean(x, axis=-1, keepdims=True)
    xc = x - mu
    var = jnp.mean(xc * xc, axis=-1, keepdims=True)
    o_ref[...] = xc * lax.rsqrt(var + _EPS) * scale_ref[...] + bias_ref[...]


def _tc_fuse(gathered, cond2d, cond_W, cond_b, ln_scale, ln_bias):
    grid = (_NTOK // _TBLK,)
    return pl.pallas_call(
        _tc_body,
        grid=grid,
        in_specs=[
            pl.BlockSpec((_TBLK, _EMB), lambda i: (i, 0)),
            pl.BlockSpec((_TBLK, _COND), lambda i: (i, 0)),
            pl.BlockSpec((_COND, _EMB), lambda i: (0, 0)),
            pl.BlockSpec((1, _EMB), lambda i: (0, 0)),
            pl.BlockSpec((1, _EMB), lambda i: (0, 0)),
            pl.BlockSpec((1, _EMB), lambda i: (0, 0)),
        ],
        out_specs=pl.BlockSpec((_TBLK, _EMB), lambda i: (i, 0)),
        out_shape=jax.ShapeDtypeStruct((_NTOK, _EMB), jnp.float32),
    )(gathered, cond2d, cond_W,
      cond_b.reshape(1, _EMB), ln_scale.reshape(1, _EMB), ln_bias.reshape(1, _EMB))


def kernel(ids, conditions, pos_ids, ids_table, pos_table, cond_W, cond_b,
           ln_scale, ln_bias):
    ids_flat = ids.reshape(_NTOK)
    pos_flat = pos_ids.reshape(_NTOK)
    gathered = _sc_gather_sum(ids_flat, pos_flat, ids_table, pos_table)
    y = _tc_fuse(gathered, conditions.reshape(_NTOK, _COND), cond_W, cond_b,
                 ln_scale, ln_bias)
    return y.reshape(_B, _S, _EMB)


# speedup vs baseline: 2.1134x; 1.0731x over previous
import functools

import jax
import jax.numpy as jnp
from jax import lax
from jax.experimental import pallas as pl
from jax.experimental.pallas import tpu as pltpu
from jax.experimental.pallas import tpu_sc as plsc

_VOCAB = 100000
_MAXLEN = 2048
_EMB = 768
_COND = 128
_B = 4
_S = 2048
_EPS = 1e-12

_NTOK = _B * _S
_NC = 2
_NS = 16
_NW = _NC * _NS
_PER_W = _NTOK // _NW
_C = 16
_NCHUNK = _PER_W // _C
_LANES = 16


def _sc_gather_sum(ids_flat, pos_flat, ids_table, pos_table):
    mesh = plsc.VectorSubcoreMesh(core_axis_name="c", subcore_axis_name="s")

    @functools.partial(
        pl.kernel,
        mesh=mesh,
        out_type=jax.ShapeDtypeStruct((_NTOK, _EMB), jnp.float32),
        scratch_types=[
            pltpu.VMEM((_PER_W,), jnp.int32),
            pltpu.VMEM((_PER_W,), jnp.int32),
            pltpu.VMEM((2, _C, _EMB), jnp.float32),
            pltpu.VMEM((2, _C, _EMB), jnp.float32),
            pltpu.VMEM((2, _C, _EMB), jnp.float32),
            pltpu.SemaphoreType.DMA,
            pltpu.SemaphoreType.DMA,
            pltpu.SemaphoreType.DMA,
            pltpu.SemaphoreType.DMA,
            pltpu.SemaphoreType.DMA,
            pltpu.SemaphoreType.DMA,
        ],
    )
    def sc_kernel(ids_hbm, pos_hbm, idtab_hbm, postab_hbm, out_hbm,
                  idx_i, idx_p, rows_i, rows_p, rows_w,
                  sem_gi0, sem_gi1, sem_gp0, sem_gp1, sem_w0, sem_w1):
        sem_gi = (sem_gi0, sem_gi1)
        sem_gp = (sem_gp0, sem_gp1)
        sem_w = (sem_w0, sem_w1)
        wid = lax.axis_index("s") * _NC + lax.axis_index("c")
        base = wid * _PER_W

        pltpu.sync_copy(ids_hbm.at[pl.ds(base, _PER_W)], idx_i)
        pltpu.sync_copy(pos_hbm.at[pl.ds(base, _PER_W)], idx_p)

        def fire_gathers(c):
            s = c % 2
            gi = pltpu.async_copy(
                idtab_hbm.at[idx_i.at[pl.ds(c * _C, _C)]], rows_i.at[s],
                sem_gi[s])
            gp = pltpu.async_copy(
                postab_hbm.at[idx_p.at[pl.ds(c * _C, _C)]], rows_p.at[s],
                sem_gp[s])
            return gi, gp

        pend_g = [None, None]
        pend_w = [None, None]
        pend_g[0] = fire_gathers(0)

        for c in range(_NCHUNK):
            s = c % 2
            if c + 1 < _NCHUNK:
                pend_g[1 - s] = fire_gathers(c + 1)
            gi, gp = pend_g[s]
            gi.wait()
            gp.wait()
            if pend_w[s] is not None:
                pend_w[s].wait()

            def row_body(r, c2):
                for v in range(_EMB // _LANES):
                    sl = pl.ds(v * _LANES, _LANES)
                    rows_w[s, r, sl] = rows_i[s, r, sl] + rows_p[s, r, sl]
                return c2

            lax.fori_loop(0, _C, row_body, 0)
            pend_w[s] = pltpu.async_copy(
                rows_w.at[s], out_hbm.at[pl.ds(base + c * _C, _C)], sem_w[s])
        pend_w[0].wait()
        pend_w[1].wait()

    return sc_kernel(ids_flat, pos_flat, ids_table, pos_table)


_TBLK = 512


def _tc_body(g_ref, cond_ref, w_ref, b_ref, scale_ref, bias_ref, o_ref):
    proj = jnp.dot(cond_ref[...], w_ref[...], preferred_element_type=jnp.float32)
    x = g_ref[...] + jnp.tanh(proj + b_ref[...])
    mu = jnp.mean(x, axis=-1, keepdims=True)
    xc = x - mu
    var = jnp.mean(xc * xc, axis=-1, keepdims=True)
    o_ref[...] = xc * lax.rsqrt(var + _EPS) * scale_ref[...] + bias_ref[...]


def _tc_fuse(gathered, cond2d, cond_W, cond_b, ln_scale, ln_bias):
    grid = (_NTOK // _TBLK,)
    return pl.pallas_call(
        _tc_body,
        grid=grid,
        in_specs=[
            pl.BlockSpec((_TBLK, _EMB), lambda i: (i, 0)),
            pl.BlockSpec((_TBLK, _COND), lambda i: (i, 0)),
            pl.BlockSpec((_COND, _EMB), lambda i: (0, 0)),
            pl.BlockSpec((1, _EMB), lambda i: (0, 0)),
            pl.BlockSpec((1, _EMB), lambda i: (0, 0)),
            pl.BlockSpec((1, _EMB), lambda i: (0, 0)),
        ],
        out_specs=pl.BlockSpec((_TBLK, _EMB), lambda i: (i, 0)),
        out_shape=jax.ShapeDtypeStruct((_NTOK, _EMB), jnp.float32),
    )(gathered, cond2d, cond_W,
      cond_b.reshape(1, _EMB), ln_scale.reshape(1, _EMB), ln_bias.reshape(1, _EMB))


def kernel(ids, conditions, pos_ids, ids_table, pos_table, cond_W, cond_b,
           ln_scale, ln_bias):
    ids_flat = ids.reshape(_NTOK)
    pos_flat = pos_ids.reshape(_NTOK)
    gathered = _sc_gather_sum(ids_flat, pos_flat, ids_table, pos_table)
    y = _tc_fuse(gathered, conditions.reshape(_NTOK, _COND), cond_W, cond_b,
                 ln_scale, ln_bias)
    return y.reshape(_B, _S, _EMB)
